# baseline (speedup 1.0000x reference)
import functools

import jax
import jax.numpy as jnp
from jax import lax
from jax.experimental import pallas as pl
from jax.experimental.pallas import tpu as pltpu
from jax.experimental.pallas import tpu_sc as plsc

N_NODES = 10000
N_EDGES = 320000
D_FEAT = 128

NUM_CORES = 2
NUM_SUBCORES = 16
NUM_WORKERS = NUM_CORES * NUM_SUBCORES
LANES = 16
EDGES_PER_WORKER = N_EDGES // NUM_WORKERS
ITERS = EDGES_PER_WORKER // LANES


def _scores_body(x_ref, w2_ref, b_ref, st_ref):
    x = x_ref[...]
    w2 = w2_ref[...]
    st = lax.dot_general(
        w2, x, (((1,), (1,)), ((), ())),
        preferred_element_type=jnp.float32,
    )
    bias = jnp.where(
        lax.broadcasted_iota(jnp.int32, st.shape, 0) == 0, b_ref[0, 0], 0.0
    )
    st_ref[...] = st + bias


def _node_scores(node_features, W, b):
    w2 = W.reshape(2, D_FEAT)
    bb = b.reshape(1, 1)
    return pl.pallas_call(
        _scores_body,
        out_shape=jax.ShapeDtypeStruct((2, N_NODES), jnp.float32),
        in_specs=[
            pl.BlockSpec(memory_space=pltpu.VMEM),
            pl.BlockSpec(memory_space=pltpu.VMEM),
            pl.BlockSpec(memory_space=pltpu.SMEM),
        ],
        out_specs=pl.BlockSpec(memory_space=pltpu.VMEM),
    )(node_features, w2, bb)


def _edge_body(st_hbm, ei_hbm, out_hbm,
               s_tab, t_tab, src_v, dst_v, out_v, sem):
    wid = lax.axis_index("s") * NUM_CORES + lax.axis_index("c")
    base = wid * EDGES_PER_WORKER
    c1 = pltpu.async_copy(ei_hbm.at[0, pl.ds(base, EDGES_PER_WORKER)], src_v, sem)
    c2 = pltpu.async_copy(ei_hbm.at[1, pl.ds(base, EDGES_PER_WORKER)], dst_v, sem)
    c3 = pltpu.async_copy(st_hbm.at[0], s_tab, sem)
    c4 = pltpu.async_copy(st_hbm.at[1], t_tab, sem)
    c1.wait()
    c2.wait()
    c3.wait()
    c4.wait()

    @plsc.parallel_loop(0, ITERS, unroll=4)
    def body(i):
        sl = pl.ds(i * LANES, LANES)
        si = src_v[sl]
        di = dst_v[sl]
        sv = plsc.load_gather(s_tab, [si])
        tv = plsc.load_gather(t_tab, [di])
        z = sv + tv
        out_v[sl] = 1.0 / (1.0 + jnp.exp(-z))

    pltpu.sync_copy(out_v, out_hbm.at[pl.ds(base, EDGES_PER_WORKER)])


_edge_kernel = functools.partial(
    pl.kernel,
    mesh=plsc.VectorSubcoreMesh(core_axis_name="c", subcore_axis_name="s"),
    out_type=jax.ShapeDtypeStruct((N_EDGES,), jnp.float32),
    compiler_params=pltpu.CompilerParams(
        needs_layout_passes=False,
        use_tc_tiling_on_sc=False,
        disable_bounds_checks=True,
        disable_semaphore_checks=True,
    ),
    scratch_types=[
        pltpu.VMEM((N_NODES,), jnp.float32),
        pltpu.VMEM((N_NODES,), jnp.float32),
        pltpu.VMEM((EDGES_PER_WORKER,), jnp.int32),
        pltpu.VMEM((EDGES_PER_WORKER,), jnp.int32),
        pltpu.VMEM((EDGES_PER_WORKER,), jnp.float32),
        pltpu.SemaphoreType.DMA,
    ],
)(_edge_body)


def kernel(node_features, edge_index, W, b):
    st = _node_scores(node_features, W, b)
    ei = edge_index.astype(jnp.int32)
    out = _edge_kernel(st, ei)
    return out.reshape(N_EDGES, 1)


# speedup vs baseline: 1.0845x; 1.0845x over previous
import functools

import jax
import jax.numpy as jnp
from jax import lax
from jax.experimental import pallas as pl
from jax.experimental.pallas import tpu as pltpu
from jax.experimental.pallas import tpu_sc as plsc

N_NODES = 10000
N_EDGES = 320000
D_FEAT = 128

NUM_CORES = 2
NUM_SUBCORES = 16
NUM_WORKERS = NUM_CORES * NUM_SUBCORES
LANES = 16
EDGE_TILES = N_EDGES // 128
TILES_PER_WORKER = -(-EDGE_TILES // NUM_WORKERS)
CHUNKS_PER_TILE = 128 // LANES


def _scores_body(x_ref, w2_ref, b_ref, st_ref):
    x = x_ref[...]
    w2 = w2_ref[...]
    st = lax.dot_general(
        w2, x, (((1,), (1,)), ((), ())),
        preferred_element_type=jnp.float32,
    )
    bias = jnp.where(
        lax.broadcasted_iota(jnp.int32, st.shape, 0) == 0, b_ref[0, 0], 0.0
    )
    st_ref[...] = st + bias


def _node_scores(node_features, W, b):
    w2 = W.reshape(2, D_FEAT)
    bb = b.reshape(1, 1)
    return pl.pallas_call(
        _scores_body,
        out_shape=jax.ShapeDtypeStruct((2, N_NODES), jnp.float32),
        in_specs=[
            pl.BlockSpec(memory_space=pltpu.VMEM),
            pl.BlockSpec(memory_space=pltpu.VMEM),
            pl.BlockSpec(memory_space=pltpu.SMEM),
        ],
        out_specs=pl.BlockSpec(memory_space=pltpu.VMEM),
    )(node_features, w2, bb)


def _edge_body(st_hbm, ei_hbm, out_hbm, s_tab, t_tab, ei_v, out_v, sem):
    wid = lax.axis_index("s") * NUM_CORES + lax.axis_index("c")
    start = jnp.minimum(wid * TILES_PER_WORKER, EDGE_TILES - TILES_PER_WORKER)
    c1 = pltpu.async_copy(ei_hbm.at[pl.ds(start, TILES_PER_WORKER)], ei_v, sem)
    c2 = pltpu.async_copy(st_hbm.at[0], s_tab, sem)
    c3 = pltpu.async_copy(st_hbm.at[1], t_tab, sem)
    c1.wait()
    c2.wait()
    c3.wait()

    @plsc.parallel_loop(0, TILES_PER_WORKER, unroll=2)
    def body(t):
        for k in range(CHUNKS_PER_TILE):
            sl = pl.ds(k * LANES, LANES)
            si = ei_v[t, 0, sl]
            di = ei_v[t, 1, sl]
            sv = plsc.load_gather(s_tab, [si])
            tv = plsc.load_gather(t_tab, [di])
            z = sv + tv
            out_v[t, sl] = 1.0 / (1.0 + jnp.exp(-z))

    pltpu.sync_copy(out_v, out_hbm.at[pl.ds(start, TILES_PER_WORKER)])


_edge_kernel = functools.partial(
    pl.kernel,
    mesh=plsc.VectorSubcoreMesh(core_axis_name="c", subcore_axis_name="s"),
    out_type=jax.ShapeDtypeStruct((EDGE_TILES, 128), jnp.float32),
    compiler_params=pltpu.CompilerParams(
        needs_layout_passes=False,
        use_tc_tiling_on_sc=False,
        disable_bounds_checks=True,
        disable_semaphore_checks=True,
    ),
    scratch_types=[
        pltpu.VMEM((N_NODES,), jnp.float32),
        pltpu.VMEM((N_NODES,), jnp.float32),
        pltpu.VMEM((TILES_PER_WORKER, 2, 128), jnp.int32),
        pltpu.VMEM((TILES_PER_WORKER, 128), jnp.float32),
        pltpu.SemaphoreType.DMA,
    ],
)(_edge_body)


def kernel(node_features, edge_index, W, b):
    st = _node_scores(node_features, W, b)
    ei = edge_index.astype(jnp.int32)
    ei3 = ei.reshape(2, EDGE_TILES, 128).transpose(1, 0, 2)
    out = _edge_kernel(st, ei3)
    return out.reshape(N_EDGES, 1)


# speedup vs baseline: 1.1368x; 1.0482x over previous
import functools

import jax
import jax.numpy as jnp
from jax import lax
from jax.experimental import pallas as pl
from jax.experimental.pallas import tpu as pltpu
from jax.experimental.pallas import tpu_sc as plsc

N_NODES = 10000
N_NODES_PAD = 10240
N_EDGES = 320000
D_FEAT = 128

NUM_CORES = 2
NUM_SUBCORES = 16
NUM_WORKERS = NUM_CORES * NUM_SUBCORES
LANES = 16
EDGE_TILES = N_EDGES // 128
TILES_PER_WORKER = -(-EDGE_TILES // NUM_WORKERS)
CHUNKS_PER_TILE = 128 // LANES


def _scores_body(x_ref, w2_ref, b_ref, s_ref, t_ref):
    x = x_ref[...]
    w2 = w2_ref[...]
    st = lax.dot_general(
        w2, x, (((1,), (1,)), ((), ())),
        preferred_element_type=jnp.float32,
    )
    s_ref[pl.ds(0, N_NODES)] = st[0:1, :].reshape(N_NODES) + b_ref[0, 0]
    t_ref[pl.ds(0, N_NODES)] = st[1:2, :].reshape(N_NODES)


def _node_scores(node_features, W, b):
    w2 = W.reshape(2, D_FEAT)
    bb = b.reshape(1, 1)
    return pl.pallas_call(
        _scores_body,
        out_shape=(
            jax.ShapeDtypeStruct((N_NODES_PAD,), jnp.float32),
            jax.ShapeDtypeStruct((N_NODES_PAD,), jnp.float32),
        ),
        in_specs=[
            pl.BlockSpec(memory_space=pltpu.VMEM),
            pl.BlockSpec(memory_space=pltpu.VMEM),
            pl.BlockSpec(memory_space=pltpu.SMEM),
        ],
        out_specs=(
            pl.BlockSpec(memory_space=pltpu.VMEM),
            pl.BlockSpec(memory_space=pltpu.VMEM),
        ),
    )(node_features, w2, bb)


def _edge_body(s_hbm, t_hbm, ei_hbm, out_hbm, s_tab, t_tab, ei_v, out_v, sem):
    wid = lax.axis_index("s") * NUM_CORES + lax.axis_index("c")
    start = jnp.minimum(wid * TILES_PER_WORKER, EDGE_TILES - TILES_PER_WORKER)
    c1 = pltpu.async_copy(ei_hbm.at[pl.ds(start, TILES_PER_WORKER)], ei_v, sem)
    c2 = pltpu.async_copy(s_hbm, s_tab, sem)
    c3 = pltpu.async_copy(t_hbm, t_tab, sem)
    c1.wait()
    c2.wait()
    c3.wait()

    @plsc.parallel_loop(0, TILES_PER_WORKER, unroll=2)
    def body(t):
        for k in range(CHUNKS_PER_TILE):
            sl = pl.ds(k * LANES, LANES)
            si = ei_v[t, 0, sl]
            di = ei_v[t, 1, sl]
            sv = plsc.load_gather(s_tab, [si])
            tv = plsc.load_gather(t_tab, [di])
            z = sv + tv
            out_v[t, sl] = 1.0 / (1.0 + jnp.exp(-z))

    pltpu.sync_copy(out_v, out_hbm.at[pl.ds(start, TILES_PER_WORKER)])


_edge_kernel = functools.partial(
    pl.kernel,
    mesh=plsc.VectorSubcoreMesh(core_axis_name="c", subcore_axis_name="s"),
    out_type=jax.ShapeDtypeStruct((EDGE_TILES, 128), jnp.float32),
    compiler_params=pltpu.CompilerParams(
        needs_layout_passes=False,
        use_tc_tiling_on_sc=False,
        disable_bounds_checks=True,
        disable_semaphore_checks=True,
    ),
    scratch_types=[
        pltpu.VMEM((N_NODES_PAD,), jnp.float32),
        pltpu.VMEM((N_NODES_PAD,), jnp.float32),
        pltpu.VMEM((TILES_PER_WORKER, 2, 128), jnp.int32),
        pltpu.VMEM((TILES_PER_WORKER, 128), jnp.float32),
        pltpu.SemaphoreType.DMA,
    ],
)(_edge_body)


def kernel(node_features, edge_index, W, b):
    s, t = _node_scores(node_features, W, b)
    ei = edge_index.astype(jnp.int32)
    ei3 = ei.reshape(2, EDGE_TILES, 128).transpose(1, 0, 2)
    out = _edge_kernel(s, t, ei3)
    return out.reshape(N_EDGES, 1)


# speedup vs baseline: 1.1677x; 1.0272x over previous
import functools

import jax
import jax.numpy as jnp
from jax import lax
from jax.experimental import pallas as pl
from jax.experimental.pallas import tpu as pltpu
from jax.experimental.pallas import tpu_sc as plsc

N_NODES = 10000
N_NODES_PAD = 10240
N_EDGES = 320000
D_FEAT = 128

NUM_CORES = 2
NUM_SUBCORES = 16
NUM_WORKERS = NUM_CORES * NUM_SUBCORES
LANES = 16
EDGE_TILES = N_EDGES // 128
TILES_PER_WORKER = -(-EDGE_TILES // NUM_WORKERS)
CHUNKS_PER_TILE = 128 // LANES


def _scores_body(x_ref, w2_ref, b_ref, s_ref, t_ref):
    x = x_ref[...]
    w2 = w2_ref[...]
    st = lax.dot_general(
        w2, x, (((1,), (1,)), ((), ())),
        preferred_element_type=jnp.float32,
    )
    s_ref[pl.ds(0, N_NODES)] = st[0:1, :].reshape(N_NODES) + b_ref[0, 0]
    t_ref[pl.ds(0, N_NODES)] = st[1:2, :].reshape(N_NODES)


def _node_scores(node_features, W, b):
    w2 = W.reshape(2, D_FEAT)
    bb = b.reshape(1, 1)
    return pl.pallas_call(
        _scores_body,
        out_shape=(
            jax.ShapeDtypeStruct((N_NODES_PAD,), jnp.float32),
            jax.ShapeDtypeStruct((N_NODES_PAD,), jnp.float32),
        ),
        in_specs=[
            pl.BlockSpec(memory_space=pltpu.VMEM),
            pl.BlockSpec(memory_space=pltpu.VMEM),
            pl.BlockSpec(memory_space=pltpu.SMEM),
        ],
        out_specs=(
            pl.BlockSpec(memory_space=pltpu.VMEM),
            pl.BlockSpec(memory_space=pltpu.VMEM),
        ),
    )(node_features, w2, bb)


def _edge_body(s_hbm, t_hbm, ei_hbm, out_hbm, s_tab, t_tab, ei_v, out_v, sem):
    wid = lax.axis_index("s") * NUM_CORES + lax.axis_index("c")
    start = jnp.minimum(wid * TILES_PER_WORKER, EDGE_TILES - TILES_PER_WORKER)
    c1 = pltpu.async_copy(ei_hbm.at[pl.ds(start, TILES_PER_WORKER)], ei_v, sem)
    c2 = pltpu.async_copy(s_hbm, s_tab, sem)
    c3 = pltpu.async_copy(t_hbm, t_tab, sem)
    c1.wait()
    c2.wait()
    c3.wait()

    @plsc.parallel_loop(0, TILES_PER_WORKER * CHUNKS_PER_TILE, unroll=4)
    def body(c):
        t = c >> 3
        sl = pl.ds((c & 7) * LANES, LANES)
        si = ei_v[t, 0, sl]
        di = ei_v[t, 1, sl]
        sv = plsc.load_gather(s_tab, [si])
        tv = plsc.load_gather(t_tab, [di])
        z = sv + tv
        out_v[t, sl] = 1.0 / (1.0 + jnp.exp(-z))

    pltpu.sync_copy(out_v, out_hbm.at[pl.ds(start, TILES_PER_WORKER)])


_edge_kernel = functools.partial(
    pl.kernel,
    mesh=plsc.VectorSubcoreMesh(core_axis_name="c", subcore_axis_name="s"),
    out_type=jax.ShapeDtypeStruct((EDGE_TILES, 128), jnp.float32),
    compiler_params=pltpu.CompilerParams(
        needs_layout_passes=False,
        use_tc_tiling_on_sc=False,
        disable_bounds_checks=True,
        disable_semaphore_checks=True,
    ),
    scratch_types=[
        pltpu.VMEM((N_NODES_PAD,), jnp.float32),
        pltpu.VMEM((N_NODES_PAD,), jnp.float32),
        pltpu.VMEM((TILES_PER_WORKER, 2, 128), jnp.int32),
        pltpu.VMEM((TILES_PER_WORKER, 128), jnp.float32),
        pltpu.SemaphoreType.DMA,
    ],
)(_edge_body)


def kernel(node_features, edge_index, W, b):
    s, t = _node_scores(node_features, W, b)
    ei = edge_index.astype(jnp.int32)
    ei3 = ei.reshape(2, EDGE_TILES, 128).transpose(1, 0, 2)
    out = _edge_kernel(s, t, ei3)
    return out.reshape(N_EDGES, 1)
